# Initial kernel scaffold; baseline (speedup 1.0000x reference)
import jax
import jax.numpy as jnp
from jax.experimental import pallas as pl


def kernel(output, mask, ind, target):
    raise NotImplementedError("write your pallas kernel here")


# speedup vs baseline: 1.3420x; 1.3420x over previous
import jax
import jax.numpy as jnp
from jax import lax
from jax.experimental import pallas as pl
from jax.experimental.pallas import tpu as pltpu
from jax.experimental.pallas import tpu_sc as plsc

B, D, H, W, M = 32, 4, 128, 128, 128
HW = H * W
L = 16
NC = 2
NS = 16


def _tile_body(flat_hbm, mask_hbm, ind_hbm, tgt_hbm, out_hbm,
               ind_v, idx_v, pred_v, msk_v, tgt_v, part_v, sem):
    c = lax.axis_index("c")
    s = lax.axis_index("s")
    wid = s * NC + c
    b = wid

    pltpu.sync_copy(ind_hbm.at[b], ind_v)
    pltpu.sync_copy(mask_hbm.at[b], msk_v)
    pltpu.sync_copy(tgt_hbm.at[b], tgt_v)

    base = b * (D * HW)
    for d in range(D):
        off = base + d * HW
        for ch in range(M // L):
            iv = ind_v[pl.ds(ch * L, L)]
            idx_v[d, pl.ds(ch * L, L)] = iv + off

    copies = [
        pltpu.async_copy(flat_hbm.at[idx_v.at[d]], pred_v.at[d], sem)
        for d in range(D)
    ]
    for cp in copies:
        cp.wait()

    acc = jnp.zeros((L,), jnp.float32)
    macc = jnp.zeros((L,), jnp.float32)
    for ch in range(M // L):
        mv = msk_v[pl.ds(ch * L, L)].astype(jnp.float32)
        macc = macc + mv
        for d in range(D):
            p = pred_v[d, pl.ds(ch * L, L)]
            t = tgt_v[d, pl.ds(ch * L, L)]
            diff = (p - t) * mv
            a = jnp.abs(diff)
            acc = acc + jnp.where(a < 1.0, 0.5 * diff * diff, a - 0.5)

    part_v[0, pl.ds(0, L)] = acc
    part_v[1, pl.ds(0, L)] = macc
    pltpu.sync_copy(part_v, out_hbm.at[wid])


@jax.jit
def kernel(output, mask, ind, target):
    flat = output.reshape(B * D * HW)
    tgt_t = jnp.transpose(target, (0, 2, 1))
    mesh = plsc.VectorSubcoreMesh(core_axis_name="c", subcore_axis_name="s")
    parts = pl.kernel(
        _tile_body,
        out_type=jax.ShapeDtypeStruct((NC * NS, 2, L), jnp.float32),
        mesh=mesh,
        scratch_types=[
            pltpu.VMEM((M,), jnp.int32),
            pltpu.VMEM((D, M), jnp.int32),
            pltpu.VMEM((D, M), jnp.float32),
            pltpu.VMEM((M,), jnp.int32),
            pltpu.VMEM((D, M), jnp.float32),
            pltpu.VMEM((2, L), jnp.float32),
            pltpu.SemaphoreType.DMA,
        ],
    )(flat, mask.astype(jnp.int32), ind.astype(jnp.int32), tgt_t)
    total = parts[:, 0, :].sum()
    num = parts[:, 1, :].sum()
    return total / (num + 0.0001)


# speedup vs baseline: 1.3927x; 1.0378x over previous
import jax
import jax.numpy as jnp
from jax import lax
from jax.experimental import pallas as pl
from jax.experimental.pallas import tpu as pltpu
from jax.experimental.pallas import tpu_sc as plsc

B, D, H, W, M = 32, 4, 128, 128, 128
HW = H * W
L = 16
NC = 2
NS = 16


def _tile_body(flat_hbm, mask_hbm, ind_hbm, tgt_hbm, out_hbm,
               ind_v, idx_v, pred_v, msk_v, tgt_v, part_v, sem, sem_in):
    c = lax.axis_index("c")
    s = lax.axis_index("s")
    wid = s * NC + c
    b = wid

    cp_ind = pltpu.async_copy(ind_hbm.at[b], ind_v, sem_in)
    cp_msk = pltpu.async_copy(mask_hbm.at[b], msk_v, sem_in)
    cp_tgt = pltpu.async_copy(tgt_hbm.at[b], tgt_v, sem_in)
    cp_ind.wait()

    base = b * (D * HW)
    for d in range(D):
        off = base + d * HW
        for ch in range(M // L):
            iv = ind_v[pl.ds(ch * L, L)]
            idx_v[d, pl.ds(ch * L, L)] = iv + off

    copies = [
        pltpu.async_copy(flat_hbm.at[idx_v.at[d]], pred_v.at[d], sem)
        for d in range(D)
    ]
    cp_msk.wait()
    cp_tgt.wait()
    for cp in copies:
        cp.wait()

    acc = jnp.zeros((L,), jnp.float32)
    macc = jnp.zeros((L,), jnp.float32)
    for ch in range(M // L):
        mv = msk_v[pl.ds(ch * L, L)].astype(jnp.float32)
        macc = macc + mv
        for d in range(D):
            p = pred_v[d, pl.ds(ch * L, L)]
            t = tgt_v[d, pl.ds(ch * L, L)]
            diff = (p - t) * mv
            a = jnp.abs(diff)
            acc = acc + jnp.where(a < 1.0, 0.5 * diff * diff, a - 0.5)

    part_v[0, pl.ds(0, L)] = acc
    part_v[1, pl.ds(0, L)] = macc
    pltpu.sync_copy(part_v, out_hbm.at[wid])


@jax.jit
def kernel(output, mask, ind, target):
    flat = output.reshape(B * D * HW)
    tgt_t = jnp.transpose(target, (0, 2, 1))
    mesh = plsc.VectorSubcoreMesh(core_axis_name="c", subcore_axis_name="s")
    parts = pl.kernel(
        _tile_body,
        out_type=jax.ShapeDtypeStruct((NC * NS, 2, L), jnp.float32),
        mesh=mesh,
        scratch_types=[
            pltpu.VMEM((M,), jnp.int32),
            pltpu.VMEM((D, M), jnp.int32),
            pltpu.VMEM((D, M), jnp.float32),
            pltpu.VMEM((M,), jnp.int32),
            pltpu.VMEM((D, M), jnp.float32),
            pltpu.VMEM((2, L), jnp.float32),
            pltpu.SemaphoreType.DMA,
            pltpu.SemaphoreType.DMA,
        ],
    )(flat, mask.astype(jnp.int32), ind.astype(jnp.int32), tgt_t)
    total = parts[:, 0, :].sum()
    num = parts[:, 1, :].sum()
    return total / (num + 0.0001)
